# baseline (device time: 45598 ns/iter reference)
import jax
import jax.numpy as jnp
from jax import lax
from jax.experimental import pallas as pl
from jax.experimental.pallas import tpu as pltpu

N_DEV = 32
PLANE = 8
EPS = 1e-5


def kernel(x, Wp):
    b, hs, w, c = x.shape
    c_out = Wp.shape[1]
    n_global = hs * N_DEV * w

    def body(x_ref, wp_ref, out_ref, c1, c2, c3,
             s1, r1, s2, r2, s3, r3):
        my_pos = lax.axis_index("i")
        q = lax.rem(my_pos, PLANE)
        plane_base = my_pos - q
        p1_partner = jnp.bitwise_xor(my_pos, 1)

        barrier_sem = pltpu.get_barrier_semaphore()
        partners = [p1_partner]
        partners += [plane_base + lax.rem(q + 2 * j, PLANE) for j in (1, 2, 3)]
        partners += [lax.rem(my_pos + PLANE * k, N_DEV) for k in (1, 2, 3)]
        for peer in partners:
            pl.semaphore_signal(
                barrier_sem, inc=1,
                device_id=(peer,), device_id_type=pl.DeviceIdType.MESH,
            )

        xv = x_ref[...]
        c1[0, 0] = jnp.sum(xv, axis=(1, 2))
        c1[0, 1] = jnp.sum(xv * xv, axis=(1, 2))

        pl.semaphore_wait(barrier_sem, len(partners))

        rdma = pltpu.make_async_remote_copy(
            src_ref=c1.at[0], dst_ref=c1.at[1],
            send_sem=s1, recv_sem=r1,
            device_id=(p1_partner,), device_id_type=pl.DeviceIdType.MESH,
        )
        rdma.start()
        rdma.wait()
        c2[0] = c1[0] + c1[1]

        rdmas = []
        for j in (1, 2, 3):
            target = plane_base + lax.rem(q + 2 * j, PLANE)
            rd = pltpu.make_async_remote_copy(
                src_ref=c2.at[0], dst_ref=c2.at[j],
                send_sem=s2.at[j], recv_sem=r2.at[j],
                device_id=(target,), device_id_type=pl.DeviceIdType.MESH,
            )
            rd.start()
            rdmas.append(rd)
        for rd in rdmas:
            rd.wait()
        c3[0] = (c2[0] + c2[1]) + (c2[2] + c2[3])

        rdmas = []
        for k in (1, 2, 3):
            target = lax.rem(my_pos + PLANE * k, N_DEV)
            rd = pltpu.make_async_remote_copy(
                src_ref=c3.at[0], dst_ref=c3.at[k],
                send_sem=s3.at[k], recv_sem=r3.at[k],
                device_id=(target,), device_id_type=pl.DeviceIdType.MESH,
            )
            rd.start()
            rdmas.append(rd)
        for rd in rdmas:
            rd.wait()
        stats = (c3[0] + c3[1]) + (c3[2] + c3[3])

        mean = stats[0] / n_global
        var = stats[1] / n_global - mean * mean
        inv = lax.rsqrt(var + EPS)

        h = (xv - mean[:, None, None, :]) * inv[:, None, None, :]
        a = h * jax.nn.sigmoid(h)
        out2d = jnp.dot(
            a.reshape(b * hs * w, c), wp_ref[...],
            preferred_element_type=jnp.float32,
        )
        out_ref[...] = out2d.reshape(b, hs, w, c_out)

    return pl.pallas_call(
        body,
        out_shape=jax.ShapeDtypeStruct((b, hs, w, c_out), jnp.float32),
        in_specs=[
            pl.BlockSpec(memory_space=pltpu.VMEM),
            pl.BlockSpec(memory_space=pltpu.VMEM),
        ],
        out_specs=pl.BlockSpec(memory_space=pltpu.VMEM),
        scratch_shapes=[
            pltpu.VMEM((2, 2, b, c), jnp.float32),
            pltpu.VMEM((4, 2, b, c), jnp.float32),
            pltpu.VMEM((4, 2, b, c), jnp.float32),
            pltpu.SemaphoreType.DMA,
            pltpu.SemaphoreType.DMA,
            pltpu.SemaphoreType.DMA((4,)),
            pltpu.SemaphoreType.DMA((4,)),
            pltpu.SemaphoreType.DMA((4,)),
            pltpu.SemaphoreType.DMA((4,)),
        ],
        compiler_params=pltpu.CompilerParams(collective_id=0),
    )(x, Wp)


# device time: 43698 ns/iter; 1.0435x vs baseline; 1.0435x over previous
import jax
import jax.numpy as jnp
from jax import lax
from jax.experimental import pallas as pl
from jax.experimental.pallas import tpu as pltpu

N_DEV = 32
EPS = 1e-5


def kernel(x, Wp):
    b, hs, w, c = x.shape
    c_out = Wp.shape[1]
    n_global = hs * N_DEV * w

    def body(x_ref, wp_ref, out_ref, comm_ref, send_sems, recv_sems):
        my_pos = lax.axis_index("i")

        barrier_sem = pltpu.get_barrier_semaphore()
        for d in range(1, N_DEV):
            peer = lax.rem(my_pos + d, N_DEV)
            pl.semaphore_signal(
                barrier_sem, inc=1,
                device_id=(peer,), device_id_type=pl.DeviceIdType.MESH,
            )
        xv = x_ref[...]
        comm_ref[0, 0] = jnp.sum(xv, axis=(1, 2))
        comm_ref[0, 1] = jnp.sum(xv * xv, axis=(1, 2))

        pl.semaphore_wait(barrier_sem, N_DEV - 1)

        rdmas = []
        for d in range(1, N_DEV):
            target = lax.rem(my_pos + d, N_DEV)
            rdma = pltpu.make_async_remote_copy(
                src_ref=comm_ref.at[0],
                dst_ref=comm_ref.at[d],
                send_sem=send_sems.at[d],
                recv_sem=recv_sems.at[d],
                device_id=(target,),
                device_id_type=pl.DeviceIdType.MESH,
            )
            rdma.start()
            rdmas.append(rdma)
        for rdma in rdmas:
            rdma.wait()

        stats = jnp.sum(comm_ref[...], axis=0)
        mean = stats[0] / n_global
        var = stats[1] / n_global - mean * mean
        inv = lax.rsqrt(var + EPS)

        h = (xv - mean[:, None, None, :]) * inv[:, None, None, :]
        a = h * jax.nn.sigmoid(h)
        out2d = jnp.dot(
            a.reshape(b * hs * w, c), wp_ref[...],
            preferred_element_type=jnp.float32,
        )
        out_ref[...] = out2d.reshape(b, hs, w, c_out)

    return pl.pallas_call(
        body,
        out_shape=jax.ShapeDtypeStruct((b, hs, w, c_out), jnp.float32),
        in_specs=[
            pl.BlockSpec(memory_space=pltpu.VMEM),
            pl.BlockSpec(memory_space=pltpu.VMEM),
        ],
        out_specs=pl.BlockSpec(memory_space=pltpu.VMEM),
        scratch_shapes=[
            pltpu.VMEM((N_DEV, 2, b, c), jnp.float32),
            pltpu.SemaphoreType.DMA((N_DEV,)),
            pltpu.SemaphoreType.DMA((N_DEV,)),
        ],
        compiler_params=pltpu.CompilerParams(collective_id=0),
    )(x, Wp)


# device time: 42358 ns/iter; 1.0765x vs baseline; 1.0316x over previous
import jax
import jax.numpy as jnp
from jax import lax
from jax.experimental import pallas as pl
from jax.experimental.pallas import tpu as pltpu

N_DEV = 32
EPS = 1e-5
N_CHUNK = 4


def kernel(x, Wp):
    b, hs, w, c = x.shape
    c_out = Wp.shape[1]
    n_global = hs * N_DEV * w
    ch = hs // N_CHUNK

    def body(x_hbm, wp_ref, out_hbm, xb, ob, comm_ref,
             send_sems, recv_sems, load_sems, store_sems):
        my_pos = lax.axis_index("i")

        barrier_sem = pltpu.get_barrier_semaphore()
        for d in range(1, N_DEV):
            peer = lax.rem(my_pos + d, N_DEV)
            pl.semaphore_signal(
                barrier_sem, inc=1,
                device_id=(peer,), device_id_type=pl.DeviceIdType.MESH,
            )

        loads = []
        for i in range(N_CHUNK):
            cp = pltpu.make_async_copy(
                x_hbm.at[:, pl.ds(i * ch, ch)],
                xb.at[:, pl.ds(i * ch, ch)],
                load_sems.at[i],
            )
            cp.start()
            loads.append(cp)

        s1 = jnp.zeros((b, c), jnp.float32)
        s2 = jnp.zeros((b, c), jnp.float32)
        for i in range(N_CHUNK):
            loads[i].wait()
            xv = xb[:, pl.ds(i * ch, ch)]
            s1 = s1 + jnp.sum(xv, axis=(1, 2))
            s2 = s2 + jnp.sum(xv * xv, axis=(1, 2))
        comm_ref[0, 0] = s1
        comm_ref[0, 1] = s2

        pl.semaphore_wait(barrier_sem, N_DEV - 1)

        rdmas = []
        for d in range(1, N_DEV):
            target = lax.rem(my_pos + d, N_DEV)
            rdma = pltpu.make_async_remote_copy(
                src_ref=comm_ref.at[0],
                dst_ref=comm_ref.at[d],
                send_sem=send_sems.at[d],
                recv_sem=recv_sems.at[d],
                device_id=(target,),
                device_id_type=pl.DeviceIdType.MESH,
            )
            rdma.start()
            rdmas.append(rdma)
        for rdma in rdmas:
            rdma.wait()

        stats = jnp.sum(comm_ref[...], axis=0)
        mean = stats[0] / n_global
        var = stats[1] / n_global - mean * mean
        inv = lax.rsqrt(var + EPS)

        stores = [None, None]
        for i in range(N_CHUNK):
            slot = i % 2
            if stores[slot] is not None:
                stores[slot].wait()
            xv = xb[:, pl.ds(i * ch, ch)]
            h = (xv - mean[:, None, None, :]) * inv[:, None, None, :]
            a = h * jax.nn.sigmoid(h)
            out2d = jnp.dot(
                a.reshape(b * ch * w, c), wp_ref[...],
                preferred_element_type=jnp.float32,
            )
            ob[slot] = out2d.reshape(b, ch, w, c_out)
            st = pltpu.make_async_copy(
                ob.at[slot],
                out_hbm.at[:, pl.ds(i * ch, ch)],
                store_sems.at[slot],
            )
            st.start()
            stores[slot] = st
        for st in stores:
            st.wait()

    return pl.pallas_call(
        body,
        out_shape=jax.ShapeDtypeStruct((b, hs, w, c_out), jnp.float32),
        in_specs=[
            pl.BlockSpec(memory_space=pl.ANY),
            pl.BlockSpec(memory_space=pltpu.VMEM),
        ],
        out_specs=pl.BlockSpec(memory_space=pl.ANY),
        scratch_shapes=[
            pltpu.VMEM((b, hs, w, c), jnp.float32),
            pltpu.VMEM((2, b, hs // N_CHUNK, w, c_out), jnp.float32),
            pltpu.VMEM((N_DEV, 2, b, c), jnp.float32),
            pltpu.SemaphoreType.DMA((N_DEV,)),
            pltpu.SemaphoreType.DMA((N_DEV,)),
            pltpu.SemaphoreType.DMA((N_CHUNK,)),
            pltpu.SemaphoreType.DMA((2,)),
        ],
        compiler_params=pltpu.CompilerParams(collective_id=0),
    )(x, Wp)


# device time: 39045 ns/iter; 1.1678x vs baseline; 1.0849x over previous
import jax
import jax.numpy as jnp
from jax import lax
from jax.experimental import pallas as pl
from jax.experimental.pallas import tpu as pltpu

N_DEV = 32
EPS = 1e-5
N_CHUNK = 4


def kernel(x, Wp):
    b, hs, w, c = x.shape
    c_out = Wp.shape[1]
    n_global = hs * N_DEV * w
    ch = hs // N_CHUNK

    def body(x_ref, wp_ref, out_ref, comm_ref, send_sems, recv_sems):
        my_pos = lax.axis_index("i")

        barrier_sem = pltpu.get_barrier_semaphore()
        for d in range(1, N_DEV):
            peer = lax.rem(my_pos + d, N_DEV)
            pl.semaphore_signal(
                barrier_sem, inc=1,
                device_id=(peer,), device_id_type=pl.DeviceIdType.MESH,
            )

        xv = x_ref[...]
        comm_ref[0, 0] = jnp.sum(xv, axis=(1, 3))
        comm_ref[0, 1] = jnp.sum(xv * xv, axis=(1, 3))

        pl.semaphore_wait(barrier_sem, N_DEV - 1)

        rdmas = []
        for d in range(1, N_DEV):
            target = lax.rem(my_pos + d, N_DEV)
            rdma = pltpu.make_async_remote_copy(
                src_ref=comm_ref.at[0],
                dst_ref=comm_ref.at[d],
                send_sem=send_sems.at[d],
                recv_sem=recv_sems.at[d],
                device_id=(target,),
                device_id_type=pl.DeviceIdType.MESH,
            )
            rdma.start()
            rdmas.append(rdma)
        for rdma in rdmas:
            rdma.wait()

        stats = jnp.sum(comm_ref[...], axis=0)
        mean = stats[0] / n_global
        var = stats[1] / n_global - mean * mean
        inv = lax.rsqrt(var + EPS)
        mean_b = mean[:, None, :, None]
        inv_b = inv[:, None, :, None]

        for i in range(N_CHUNK):
            sl = pl.ds(i * ch, ch)
            xc = x_ref[:, sl]
            h = (xc - mean_b) * inv_b
            a = h * jax.nn.sigmoid(h)
            at = jnp.transpose(a, (0, 1, 3, 2))
            out2d = jnp.dot(
                at.reshape(b * ch * w, c), wp_ref[...],
                preferred_element_type=jnp.float32,
            )
            out_ref[:, sl] = out2d.reshape(b, ch, w, c_out)

    xt = jnp.transpose(x, (0, 1, 3, 2))
    return pl.pallas_call(
        body,
        out_shape=jax.ShapeDtypeStruct((b, hs, w, c_out), jnp.float32),
        in_specs=[
            pl.BlockSpec(memory_space=pltpu.VMEM),
            pl.BlockSpec(memory_space=pltpu.VMEM),
        ],
        out_specs=pl.BlockSpec(memory_space=pltpu.VMEM),
        scratch_shapes=[
            pltpu.VMEM((N_DEV, 2, b, c), jnp.float32),
            pltpu.SemaphoreType.DMA((N_DEV,)),
            pltpu.SemaphoreType.DMA((N_DEV,)),
        ],
        compiler_params=pltpu.CompilerParams(collective_id=0),
    )(xt, Wp)


# device time: 38436 ns/iter; 1.1863x vs baseline; 1.0158x over previous
import jax
import jax.numpy as jnp
from jax import lax
from jax.experimental import pallas as pl
from jax.experimental.pallas import tpu as pltpu

N_DEV = 32
EPS = 1e-5
N_CHUNK = 4


def kernel(x, Wp):
    b, hs, w, c = x.shape
    c_out = Wp.shape[1]
    n_global = hs * N_DEV * w
    ch = hs // N_CHUNK

    def body(x_hbm, wp_ref, out_hbm, xb, ob, comm_ref,
             send_sems, recv_sems, load_sems, store_sems):
        my_pos = lax.axis_index("i")

        barrier_sem = pltpu.get_barrier_semaphore()
        for d in range(1, N_DEV):
            peer = lax.rem(my_pos + d, N_DEV)
            pl.semaphore_signal(
                barrier_sem, inc=1,
                device_id=(peer,), device_id_type=pl.DeviceIdType.MESH,
            )

        loads = []
        for i in range(N_CHUNK):
            cp = pltpu.make_async_copy(
                x_hbm.at[:, pl.ds(i * ch, ch)],
                xb.at[:, pl.ds(i * ch, ch)],
                load_sems.at[i],
            )
            cp.start()
            loads.append(cp)

        s1 = jnp.zeros((b, c), jnp.float32)
        s2 = jnp.zeros((b, c), jnp.float32)
        for i in range(N_CHUNK):
            loads[i].wait()
            xv = xb[:, pl.ds(i * ch, ch)]
            s1 = s1 + jnp.sum(xv, axis=(1, 3))
            s2 = s2 + jnp.sum(xv * xv, axis=(1, 3))
        comm_ref[0, 0] = s1
        comm_ref[0, 1] = s2

        pl.semaphore_wait(barrier_sem, N_DEV - 1)

        rdmas = []
        for d in range(1, N_DEV):
            target = lax.rem(my_pos + d, N_DEV)
            rdma = pltpu.make_async_remote_copy(
                src_ref=comm_ref.at[0],
                dst_ref=comm_ref.at[d],
                send_sem=send_sems.at[d],
                recv_sem=recv_sems.at[d],
                device_id=(target,),
                device_id_type=pl.DeviceIdType.MESH,
            )
            rdma.start()
            rdmas.append(rdma)
        for rdma in rdmas:
            rdma.wait()

        stats = jnp.sum(comm_ref[...], axis=0)
        mean = stats[0] / n_global
        var = stats[1] / n_global - mean * mean
        inv = lax.rsqrt(var + EPS)
        mean_b = mean[:, None, :, None]
        inv_b = inv[:, None, :, None]

        stores = [None, None]
        for i in range(N_CHUNK):
            slot = i % 2
            if stores[slot] is not None:
                stores[slot].wait()
            xc = xb[:, pl.ds(i * ch, ch)]
            h = (xc - mean_b) * inv_b
            a = h * jax.nn.sigmoid(h)
            at = jnp.transpose(a, (0, 1, 3, 2))
            out2d = jnp.dot(
                at.reshape(b * ch * w, c), wp_ref[...],
                preferred_element_type=jnp.float32,
            )
            ob[slot] = out2d.reshape(b, ch, w, c_out)
            st = pltpu.make_async_copy(
                ob.at[slot],
                out_hbm.at[:, pl.ds(i * ch, ch)],
                store_sems.at[slot],
            )
            st.start()
            stores[slot] = st
        for st in stores:
            st.wait()

    xt = jnp.transpose(x, (0, 1, 3, 2))
    return pl.pallas_call(
        body,
        out_shape=jax.ShapeDtypeStruct((b, hs, w, c_out), jnp.float32),
        in_specs=[
            pl.BlockSpec(memory_space=pltpu.MemorySpace.HBM),
            pl.BlockSpec(memory_space=pltpu.VMEM),
        ],
        out_specs=pl.BlockSpec(memory_space=pltpu.MemorySpace.HBM),
        scratch_shapes=[
            pltpu.VMEM((b, hs, c, w), jnp.float32),
            pltpu.VMEM((2, b, hs // N_CHUNK, w, c_out), jnp.float32),
            pltpu.VMEM((N_DEV, 2, b, c), jnp.float32),
            pltpu.SemaphoreType.DMA((N_DEV,)),
            pltpu.SemaphoreType.DMA((N_DEV,)),
            pltpu.SemaphoreType.DMA((N_CHUNK,)),
            pltpu.SemaphoreType.DMA((2,)),
        ],
        compiler_params=pltpu.CompilerParams(collective_id=0),
    )(xt, Wp)


# device time: 38368 ns/iter; 1.1884x vs baseline; 1.0018x over previous
import jax
import jax.numpy as jnp
from jax import lax
from jax.experimental import pallas as pl
from jax.experimental.pallas import tpu as pltpu

N_DEV = 32
EPS = 1e-5
N_CHUNK = 4


def kernel(x, Wp):
    b, hs, w, c = x.shape
    c_out = Wp.shape[1]
    n_global = hs * N_DEV * w
    ch = hs // N_CHUNK

    def body(x_hbm, wp_ref, out_hbm, xb, ob, comm_ref,
             send_sems, recv_sems, load_sems, store_sems):
        my_pos = lax.axis_index("i")

        barrier_sem = pltpu.get_barrier_semaphore()
        for d in range(1, N_DEV):
            peer = lax.rem(my_pos + d, N_DEV)
            pl.semaphore_signal(
                barrier_sem, inc=1,
                device_id=(peer,), device_id_type=pl.DeviceIdType.MESH,
            )

        loads = []
        for i in range(N_CHUNK):
            cp = pltpu.make_async_copy(
                x_hbm.at[:, pl.ds(i * ch, ch)],
                xb.at[:, pl.ds(i * ch, ch)],
                load_sems.at[i],
            )
            cp.start()
            loads.append(cp)

        s1 = jnp.zeros((b, c), jnp.float32)
        s2 = jnp.zeros((b, c), jnp.float32)
        for i in range(N_CHUNK):
            loads[i].wait()
            xv = xb[:, pl.ds(i * ch, ch)]
            s1 = s1 + jnp.sum(xv, axis=(1, 3))
            s2 = s2 + jnp.sum(xv * xv, axis=(1, 3))
        comm_ref[0, 0] = s1
        comm_ref[0, 1] = s2

        pl.semaphore_wait(barrier_sem, N_DEV - 1)

        rdmas = []
        for d in range(1, N_DEV):
            target = lax.rem(my_pos + d, N_DEV)
            rdma = pltpu.make_async_remote_copy(
                src_ref=comm_ref.at[0],
                dst_ref=comm_ref.at[d],
                send_sem=send_sems.at[d],
                recv_sem=recv_sems.at[d],
                device_id=(target,),
                device_id_type=pl.DeviceIdType.MESH,
            )
            rdma.start()
            rdmas.append(rdma)
        for rdma in rdmas:
            rdma.wait()

        stats = jnp.sum(comm_ref[...], axis=0)
        mean = stats[0] / n_global
        var = stats[1] / n_global - mean * mean
        inv = lax.rsqrt(var + EPS)
        mean_b = mean[:, None, :, None]
        inv_b = inv[:, None, :, None]

        stores = [None, None]
        for i in range(N_CHUNK):
            slot = i % 2
            if stores[slot] is not None:
                stores[slot].wait()
            xc = xb[:, pl.ds(i * ch, ch)]
            h = (xc - mean_b) * inv_b
            a = h * jax.nn.sigmoid(h)
            at = jnp.transpose(a, (0, 1, 3, 2))
            out2d = jnp.dot(
                at.reshape(b * ch * w, c), wp_ref[...],
                preferred_element_type=jnp.float32,
            )
            ob[slot] = out2d.reshape(b, ch, w, c_out)
            st = pltpu.make_async_copy(
                ob.at[slot],
                out_hbm.at[:, pl.ds(i * ch, ch)],
                store_sems.at[slot],
            )
            st.start()
            stores[slot] = st
        for st in stores:
            st.wait()

    xt = jnp.transpose(x, (0, 1, 3, 2))
    return pl.pallas_call(
        body,
        out_shape=jax.ShapeDtypeStruct((b, hs, w, c_out), jnp.float32),
        in_specs=[
            pl.BlockSpec(memory_space=pl.ANY),
            pl.BlockSpec(memory_space=pltpu.VMEM),
        ],
        out_specs=pl.BlockSpec(memory_space=pl.ANY),
        scratch_shapes=[
            pltpu.VMEM((b, hs, c, w), jnp.float32),
            pltpu.VMEM((2, b, hs // N_CHUNK, w, c_out), jnp.float32),
            pltpu.VMEM((N_DEV, 2, b, c), jnp.float32),
            pltpu.SemaphoreType.DMA((N_DEV,)),
            pltpu.SemaphoreType.DMA((N_DEV,)),
            pltpu.SemaphoreType.DMA((N_CHUNK,)),
            pltpu.SemaphoreType.DMA((2,)),
        ],
        compiler_params=pltpu.CompilerParams(collective_id=0),
    )(xt, Wp)


# device time: 37616 ns/iter; 1.2122x vs baseline; 1.0200x over previous
import jax
import jax.numpy as jnp
from jax import lax
from jax.experimental import pallas as pl
from jax.experimental.pallas import tpu as pltpu

N_DEV = 32
EPS = 1e-5
N_CHUNK = 4


def kernel(x, Wp):
    b, hs, w, c = x.shape
    c_out = Wp.shape[1]
    n_global = hs * N_DEV * w
    ch = hs // N_CHUNK

    def body(x_ref, wp_ref, out_hbm, ob, comm_ref,
             send_sems, recv_sems, store_sems):
        my_pos = lax.axis_index("i")

        barrier_sem = pltpu.get_barrier_semaphore()
        for d in range(1, N_DEV):
            peer = lax.rem(my_pos + d, N_DEV)
            pl.semaphore_signal(
                barrier_sem, inc=1,
                device_id=(peer,), device_id_type=pl.DeviceIdType.MESH,
            )

        xv = x_ref[...]
        comm_ref[0, 0] = jnp.sum(xv, axis=(1, 3))
        comm_ref[0, 1] = jnp.sum(xv * xv, axis=(1, 3))

        pl.semaphore_wait(barrier_sem, N_DEV - 1)

        rdmas = []
        for d in range(1, N_DEV):
            target = lax.rem(my_pos + d, N_DEV)
            rdma = pltpu.make_async_remote_copy(
                src_ref=comm_ref.at[0],
                dst_ref=comm_ref.at[d],
                send_sem=send_sems.at[d],
                recv_sem=recv_sems.at[d],
                device_id=(target,),
                device_id_type=pl.DeviceIdType.MESH,
            )
            rdma.start()
            rdmas.append(rdma)
        for rdma in rdmas:
            rdma.wait()

        stats = jnp.sum(comm_ref[...], axis=0)
        mean = stats[0] / n_global
        var = stats[1] / n_global - mean * mean
        inv = lax.rsqrt(var + EPS)
        mean_b = mean[:, None, :, None]
        inv_b = inv[:, None, :, None]

        stores = [None, None]
        for i in range(N_CHUNK):
            slot = i % 2
            if stores[slot] is not None:
                stores[slot].wait()
            xc = x_ref[:, pl.ds(i * ch, ch)]
            h = (xc - mean_b) * inv_b
            a = h * jax.nn.sigmoid(h)
            at = jnp.transpose(a, (0, 1, 3, 2))
            out2d = jnp.dot(
                at.reshape(b * ch * w, c), wp_ref[...],
                preferred_element_type=jnp.float32,
            )
            ob[slot] = out2d.reshape(b, ch, w, c_out)
            st = pltpu.make_async_copy(
                ob.at[slot],
                out_hbm.at[:, pl.ds(i * ch, ch)],
                store_sems.at[slot],
            )
            st.start()
            stores[slot] = st
        for st in stores:
            st.wait()

    xt = jnp.transpose(x, (0, 1, 3, 2))
    return pl.pallas_call(
        body,
        out_shape=jax.ShapeDtypeStruct((b, hs, w, c_out), jnp.float32),
        in_specs=[
            pl.BlockSpec(memory_space=pltpu.VMEM),
            pl.BlockSpec(memory_space=pltpu.VMEM),
        ],
        out_specs=pl.BlockSpec(memory_space=pl.ANY),
        scratch_shapes=[
            pltpu.VMEM((2, b, hs // N_CHUNK, w, c_out), jnp.float32),
            pltpu.VMEM((N_DEV, 2, b, c), jnp.float32),
            pltpu.SemaphoreType.DMA((N_DEV,)),
            pltpu.SemaphoreType.DMA((N_DEV,)),
            pltpu.SemaphoreType.DMA((2,)),
        ],
        compiler_params=pltpu.CompilerParams(collective_id=0),
    )(xt, Wp)
